# baseline (speedup 1.0000x reference)
import functools

import jax
import jax.numpy as jnp
from jax import lax
from jax.experimental import pallas as pl
from jax.experimental.pallas import tpu as pltpu
from jax.experimental.pallas import tpu_sc as plsc

_N = 5000
_LANES = 16
_NTILES = 16
_NPAD = 5120
_PER = _NPAD // _NTILES
_IMG_W = 2048.0
_IMG_H = 2048.0
_IOU_THRESH = 0.5
_SCORE_THRESH = 0.05
_MAX_DET = 4
_MAX_COORD = max(_IMG_W, _IMG_H) + 1.0
_NFLD = 8
_RECS = _NTILES * _NFLD


def _nms_body(b_h, sc_h, lb_h, out_h,
              bvm, lbv, x1o, y1o, x2o, y2o, sv, recv, rec_all, outs,
              sh_rec, sem):
  wid = lax.axis_index("s")

  @pl.when(lax.axis_index("c") == 0)
  def _():
    copies = [
        pltpu.make_async_copy(
            b_h.at[pl.ds(wid * (4 * _PER), 4 * _PER)], bvm, sem),
        pltpu.make_async_copy(
            sc_h.at[pl.ds(wid * _PER, _PER)], sv, sem),
        pltpu.make_async_copy(
            lb_h.at[pl.ds(wid * _PER, _PER)], lbv, sem),
    ]
    for c in copies:
      c.start()
    for c in copies:
      c.wait()

    lane = lax.iota(jnp.int32, _LANES)
    neg2 = jnp.full((_LANES,), -2.0, jnp.float32)
    zero_i = jnp.zeros((_LANES,), jnp.int32)

    def pre_body(base, carry):
      m, mi = carry
      bi = (lane + base) * 4
      bx1 = jnp.minimum(jnp.maximum(plsc.load_gather(bvm, [bi]), 0.0), _IMG_W)
      by1 = jnp.minimum(jnp.maximum(plsc.load_gather(bvm, [bi + 1]), 0.0), _IMG_H)
      bx2 = jnp.minimum(jnp.maximum(plsc.load_gather(bvm, [bi + 2]), 0.0), _IMG_W)
      by2 = jnp.minimum(jnp.maximum(plsc.load_gather(bvm, [bi + 3]), 0.0), _IMG_H)
      sl = pl.ds(base, _LANES)
      off = lbv[sl].astype(jnp.float32) * _MAX_COORD
      x1o[sl] = bx1 + off
      y1o[sl] = by1 + off
      x2o[sl] = bx2 + off
      y2o[sl] = by2 + off
      raw = sv[sl]
      s = jnp.where(raw > _SCORE_THRESH, raw, -1.0)
      sv[sl] = s
      upd = s > m
      m = jnp.where(upd, s, m)
      mi = jnp.where(upd, lane + base, mi)
      return m, mi

    m, mi = plsc.parallel_loop(
        0, _PER, _LANES, unroll=4, carry=(neg2, zero_i))(pre_body)

    mod4 = jnp.bitwise_and(lane, 3)
    grp4 = lax.shift_right_logical(lane, 2)
    full_box = jnp.where(mod4 <= 1, 0.0, jnp.where(mod4 == 2, _IMG_W, _IMG_H))

    ob_vec = jnp.zeros((_LANES,), jnp.float32)
    os_vec = jnp.zeros((_LANES,), jnp.float32)
    ol_vec = jnp.zeros((_LANES,), jnp.int32)

    def gat(vec, idx):
      return vec.at[idx].get(mode="promise_in_bounds")

    def local_argmax(m, mi):
      for k in (1, 2, 4, 8):
        idx = jnp.bitwise_xor(lane, k)
        om = gat(m, idx)
        omi = gat(mi, idx)
        take = (om > m) | ((om == m) & (omi < mi))
        m = jnp.where(take, om, m)
        mi = jnp.where(take, omi, mi)
      return m, mi

    for d in range(_MAX_DET):
      lm, lli = local_argmax(m, mi)

      par = (d % 2) * _RECS
      gmi = (lli + wid * _PER).astype(jnp.float32)
      lx1 = plsc.load_gather(x1o, [lli])
      ly1 = plsc.load_gather(y1o, [lli])
      lx2 = plsc.load_gather(x2o, [lli])
      ly2 = plsc.load_gather(y2o, [lli])
      llb = plsc.load_gather(lbv, [lli]).astype(jnp.float32)
      rec = jnp.where(lane == 0, lm,
            jnp.where(lane == 1, gmi,
            jnp.where(lane == 2, lx1,
            jnp.where(lane == 3, ly1,
            jnp.where(lane == 4, lx2,
            jnp.where(lane == 5, ly2,
            jnp.where(lane == 6, llb, 0.0)))))))
      recv[...] = rec
      pltpu.sync_copy(recv.at[pl.ds(0, _NFLD)],
                      sh_rec.at[pl.ds(par + wid * _NFLD, _NFLD)])
      plsc.subcore_barrier()
      pltpu.sync_copy(sh_rec.at[pl.ds(par, _RECS)], rec_all)

      wm = plsc.load_gather(rec_all, [lane * _NFLD + 0])
      wmi = plsc.load_gather(rec_all, [lane * _NFLD + 1])
      wx1 = plsc.load_gather(rec_all, [lane * _NFLD + 2])
      wy1 = plsc.load_gather(rec_all, [lane * _NFLD + 3])
      wx2 = plsc.load_gather(rec_all, [lane * _NFLD + 4])
      wy2 = plsc.load_gather(rec_all, [lane * _NFLD + 5])
      wlb = plsc.load_gather(rec_all, [lane * _NFLD + 6])
      for k in (1, 2, 4, 8):
        idx = jnp.bitwise_xor(lane, k)
        om = gat(wm, idx)
        omi = gat(wmi, idx)
        take = (om > wm) | ((om == wm) & (omi < wmi))
        wm = jnp.where(take, om, wm)
        wmi = jnp.where(take, omi, wmi)
        wx1 = jnp.where(take, gat(wx1, idx), wx1)
        wy1 = jnp.where(take, gat(wy1, idx), wy1)
        wx2 = jnp.where(take, gat(wx2, idx), wx2)
        wy2 = jnp.where(take, gat(wy2, idx), wy2)
        wlb = jnp.where(take, gat(wlb, idx), wlb)

      ca = (wx2 - wx1) * (wy2 - wy1)
      clbi = wlb.astype(jnp.int32)
      coff = wlb * _MAX_COORD
      cx1 = wx1 - coff
      cy1 = wy1 - coff
      cx2 = wx2 - coff
      cy2 = wy2 - coff

      badv = (((cy2.astype(jnp.int32) - cy1.astype(jnp.int32)) < 1)
              | ((cx2.astype(jnp.int32) - cx1.astype(jnp.int32)) < 1)
              | (wm < 0.0))
      boxsel = jnp.where(mod4 == 0, cx1,
                         jnp.where(mod4 == 1, cy1,
                                   jnp.where(mod4 == 2, cx2, cy2)))
      boxsel = jnp.where(badv, full_box, boxsel)
      ob_vec = jnp.where(grp4 == d, boxsel, ob_vec)
      os_vec = jnp.where(lane == d, jnp.where(wm < 0.0, 0.0, wm), os_vec)
      ol_vec = jnp.where(lane == d, jnp.where(badv, 0, clbi), ol_vec)

      if d + 1 < _MAX_DET:
        def sup_body(base, carry, wx1=wx1, wy1=wy1, wx2=wx2, wy2=wy2, ca=ca):
          m, mi = carry
          sl = pl.ds(base, _LANES)
          xo1 = x1o[sl]
          yo1 = y1o[sl]
          xo2 = x2o[sl]
          yo2 = y2o[sl]
          ltx = jnp.maximum(wx1, xo1)
          lty = jnp.maximum(wy1, yo1)
          rbx = jnp.minimum(wx2, xo2)
          rby = jnp.minimum(wy2, yo2)
          w = jnp.maximum(rbx - ltx, 0.0)
          h = jnp.maximum(rby - lty, 0.0)
          inter = w * h
          area = (xo2 - xo1) * (yo2 - yo1)
          union = jnp.maximum(ca + area - inter, 1e-9)
          s = jnp.where(inter > _IOU_THRESH * union, -1.0, sv[sl])
          sv[sl] = s
          upd = s > m
          m = jnp.where(upd, s, m)
          mi = jnp.where(upd, lane + base, mi)
          return m, mi

        m, mi = plsc.parallel_loop(
            0, _PER, _LANES, unroll=4, carry=(neg2, zero_i))(sup_body)

    @pl.when(wid == 0)
    def _():
      outs[pl.ds(0, _LANES)] = ob_vec
      outs[pl.ds(_LANES, _LANES)] = os_vec
      outs[pl.ds(2 * _LANES, _LANES)] = plsc.bitcast(ol_vec, jnp.float32)
      pltpu.sync_copy(outs, out_h)


@functools.cache
def _get_sc_kernel():
  mesh = plsc.VectorSubcoreMesh(core_axis_name="c", subcore_axis_name="s")
  f32 = jnp.float32
  return pl.kernel(
      _nms_body,
      out_type=jax.ShapeDtypeStruct((3 * _LANES,), f32),
      mesh=mesh,
      compiler_params=pltpu.CompilerParams(needs_layout_passes=False),
      scratch_types=[
          pltpu.VMEM((4 * _PER,), f32),
          pltpu.VMEM((_PER,), jnp.int32),
          pltpu.VMEM((_PER,), f32),
          pltpu.VMEM((_PER,), f32),
          pltpu.VMEM((_PER,), f32),
          pltpu.VMEM((_PER,), f32),
          pltpu.VMEM((_PER,), f32),
          pltpu.VMEM((_LANES,), f32),
          pltpu.VMEM((_RECS,), f32),
          pltpu.VMEM((3 * _LANES,), f32),
          pltpu.VMEM_SHARED((2 * _RECS,), f32),
          pltpu.SemaphoreType.DMA,
      ],
  )


def kernel(boxes, scores, labels):
  pad = _NPAD - boxes.shape[0]
  bflat = jnp.pad(boxes, ((0, pad), (0, 0))).reshape(-1)
  sc = jnp.pad(scores, (0, pad))
  lb = jnp.pad(labels, (0, pad))
  out = _get_sc_kernel()(bflat, sc, lb)
  ob = out[: _LANES].reshape(_MAX_DET, 4)
  osc = out[_LANES : _LANES + _MAX_DET]
  olb = lax.bitcast_convert_type(
      out[2 * _LANES : 2 * _LANES + _MAX_DET], jnp.int32)
  return (ob, osc, olb)


# speedup vs baseline: 1.0530x; 1.0530x over previous
import functools

import jax
import jax.numpy as jnp
from jax import lax
from jax.experimental import pallas as pl
from jax.experimental.pallas import tpu as pltpu
from jax.experimental.pallas import tpu_sc as plsc

_N = 5000
_LANES = 16
_NTILES = 16
_PER = 320
_STRIDE = 312
_IMG_W = 2048.0
_IMG_H = 2048.0
_IOU_THRESH = 0.5
_SCORE_THRESH = 0.05
_MAX_DET = 4
_MAX_COORD = max(_IMG_W, _IMG_H) + 1.0
_NFLD = 8
_RECS = _NTILES * _NFLD


def _nms_body(b_h, sc_h, lb_h, ob_h, os_h, ol_h,
              bvm, lbv, x1o, y1o, x2o, y2o, sv, recv, rec_all,
              stg_b, stg_s, stg_l, sh_rec, sem):
  wid = lax.axis_index("s")

  @pl.when(lax.axis_index("c") == 0)
  def _():
    start = wid * _STRIDE
    copies = [
        pltpu.make_async_copy(b_h.at[pl.ds(start, _PER), :], bvm, sem),
        pltpu.make_async_copy(sc_h.at[pl.ds(start, _PER)], sv, sem),
        pltpu.make_async_copy(lb_h.at[pl.ds(start, _PER)], lbv, sem),
    ]
    for c in copies:
      c.start()
    for c in copies:
      c.wait()

    lane = lax.iota(jnp.int32, _LANES)
    neg2 = jnp.full((_LANES,), -2.0, jnp.float32)
    zero_i = jnp.zeros((_LANES,), jnp.int32)
    zero_v = jnp.zeros((_LANES,), jnp.float32)

    def pre_body(base, carry):
      m, mi = carry
      ri = lane + base
      bx1 = jnp.minimum(jnp.maximum(plsc.load_gather(bvm, [ri, zero_i]), 0.0), _IMG_W)
      by1 = jnp.minimum(jnp.maximum(plsc.load_gather(bvm, [ri, zero_i + 1]), 0.0), _IMG_H)
      bx2 = jnp.minimum(jnp.maximum(plsc.load_gather(bvm, [ri, zero_i + 2]), 0.0), _IMG_W)
      by2 = jnp.minimum(jnp.maximum(plsc.load_gather(bvm, [ri, zero_i + 3]), 0.0), _IMG_H)
      sl = pl.ds(base, _LANES)
      off = lbv[sl].astype(jnp.float32) * _MAX_COORD
      x1o[sl] = bx1 + off
      y1o[sl] = by1 + off
      x2o[sl] = bx2 + off
      y2o[sl] = by2 + off
      raw = sv[sl]
      s = jnp.where(raw > _SCORE_THRESH, raw, -1.0)
      sv[sl] = s
      upd = s > m
      m = jnp.where(upd, s, m)
      mi = jnp.where(upd, ri, mi)
      return m, mi

    m, mi = plsc.parallel_loop(
        0, _PER, _LANES, unroll=4, carry=(neg2, zero_i))(pre_body)

    def gat(vec, idx):
      return vec.at[idx].get(mode="promise_in_bounds")

    def local_argmax(m, mi):
      for k in (1, 2, 4, 8):
        idx = jnp.bitwise_xor(lane, k)
        om = gat(m, idx)
        omi = gat(mi, idx)
        take = (om > m) | ((om == m) & (omi < mi))
        m = jnp.where(take, om, m)
        mi = jnp.where(take, omi, mi)
      return m, mi

    os_vec = zero_v
    ol_vec = zero_i

    for d in range(_MAX_DET):
      lm, lli = local_argmax(m, mi)

      par = (d % 2) * _RECS
      gmi = (lli + wid * _STRIDE).astype(jnp.float32)
      lx1 = plsc.load_gather(x1o, [lli])
      ly1 = plsc.load_gather(y1o, [lli])
      lx2 = plsc.load_gather(x2o, [lli])
      ly2 = plsc.load_gather(y2o, [lli])
      llb = plsc.load_gather(lbv, [lli]).astype(jnp.float32)
      rec = jnp.where(lane == 0, lm,
            jnp.where(lane == 1, gmi,
            jnp.where(lane == 2, lx1,
            jnp.where(lane == 3, ly1,
            jnp.where(lane == 4, lx2,
            jnp.where(lane == 5, ly2,
            jnp.where(lane == 6, llb, 0.0)))))))
      recv[...] = rec
      pltpu.sync_copy(recv.at[pl.ds(0, _NFLD)],
                      sh_rec.at[pl.ds(par + wid * _NFLD, _NFLD)])
      plsc.subcore_barrier()
      pltpu.sync_copy(sh_rec.at[pl.ds(par, _RECS)], rec_all)

      wm = plsc.load_gather(rec_all, [lane * _NFLD + 0])
      wmi = plsc.load_gather(rec_all, [lane * _NFLD + 1])
      wx1 = plsc.load_gather(rec_all, [lane * _NFLD + 2])
      wy1 = plsc.load_gather(rec_all, [lane * _NFLD + 3])
      wx2 = plsc.load_gather(rec_all, [lane * _NFLD + 4])
      wy2 = plsc.load_gather(rec_all, [lane * _NFLD + 5])
      wlb = plsc.load_gather(rec_all, [lane * _NFLD + 6])
      for k in (1, 2, 4, 8):
        idx = jnp.bitwise_xor(lane, k)
        om = gat(wm, idx)
        omi = gat(wmi, idx)
        take = (om > wm) | ((om == wm) & (omi < wmi))
        wm = jnp.where(take, om, wm)
        wmi = jnp.where(take, omi, wmi)
        wx1 = jnp.where(take, gat(wx1, idx), wx1)
        wy1 = jnp.where(take, gat(wy1, idx), wy1)
        wx2 = jnp.where(take, gat(wx2, idx), wx2)
        wy2 = jnp.where(take, gat(wy2, idx), wy2)
        wlb = jnp.where(take, gat(wlb, idx), wlb)

      ca = (wx2 - wx1) * (wy2 - wy1)
      clbi = wlb.astype(jnp.int32)
      coff = wlb * _MAX_COORD
      cx1 = wx1 - coff
      cy1 = wy1 - coff
      cx2 = wx2 - coff
      cy2 = wy2 - coff

      badv = (((cy2.astype(jnp.int32) - cy1.astype(jnp.int32)) < 1)
              | ((cx2.astype(jnp.int32) - cx1.astype(jnp.int32)) < 1)
              | (wm < 0.0))
      bl = jnp.where(lane == 0, cx1,
                     jnp.where(lane == 1, cy1,
                               jnp.where(lane == 2, cx2, cy2)))
      full_b = jnp.where(lane <= 1, 0.0,
                         jnp.where(lane == 2, _IMG_W, _IMG_H))
      bl = jnp.where(badv, full_b, bl)
      stg_b[pl.ds(d * _LANES, _LANES)] = bl
      os_vec = jnp.where(lane == d, jnp.where(wm < 0.0, 0.0, wm), os_vec)
      ol_vec = jnp.where(lane == d, jnp.where(badv, 0, clbi), ol_vec)

      if d + 1 < _MAX_DET:
        def sup_body(base, carry, wx1=wx1, wy1=wy1, wx2=wx2, wy2=wy2, ca=ca):
          m, mi = carry
          sl = pl.ds(base, _LANES)
          xo1 = x1o[sl]
          yo1 = y1o[sl]
          xo2 = x2o[sl]
          yo2 = y2o[sl]
          ltx = jnp.maximum(wx1, xo1)
          lty = jnp.maximum(wy1, yo1)
          rbx = jnp.minimum(wx2, xo2)
          rby = jnp.minimum(wy2, yo2)
          w = jnp.maximum(rbx - ltx, 0.0)
          h = jnp.maximum(rby - lty, 0.0)
          inter = w * h
          area = (xo2 - xo1) * (yo2 - yo1)
          union = jnp.maximum(ca + area - inter, 1e-9)
          s = jnp.where(inter > _IOU_THRESH * union, -1.0, sv[sl])
          sv[sl] = s
          upd = s > m
          m = jnp.where(upd, s, m)
          mi = jnp.where(upd, lane + base, mi)
          return m, mi

        m, mi = plsc.parallel_loop(
            0, _PER, _LANES, unroll=4, carry=(neg2, zero_i))(sup_body)

    @pl.when(wid == 0)
    def _():
      stg_s[...] = os_vec
      stg_l[...] = ol_vec
      outc = [
          pltpu.make_async_copy(stg_b, ob_h, sem),
          pltpu.make_async_copy(stg_s.at[pl.ds(0, _MAX_DET)], os_h, sem),
          pltpu.make_async_copy(stg_l.at[pl.ds(0, _MAX_DET)], ol_h, sem),
      ]
      for c in outc:
        c.start()
      for c in outc:
        c.wait()


@functools.cache
def _get_sc_kernel():
  mesh = plsc.VectorSubcoreMesh(core_axis_name="c", subcore_axis_name="s")
  f32 = jnp.float32
  return pl.kernel(
      _nms_body,
      out_type=(
          jax.ShapeDtypeStruct((_MAX_DET * _LANES,), f32),
          jax.ShapeDtypeStruct((_MAX_DET,), f32),
          jax.ShapeDtypeStruct((_MAX_DET,), jnp.int32),
      ),
      mesh=mesh,
      compiler_params=pltpu.CompilerParams(needs_layout_passes=False),
      scratch_types=[
          pltpu.VMEM((_PER, 4), f32),
          pltpu.VMEM((_PER,), jnp.int32),
          pltpu.VMEM((_PER,), f32),
          pltpu.VMEM((_PER,), f32),
          pltpu.VMEM((_PER,), f32),
          pltpu.VMEM((_PER,), f32),
          pltpu.VMEM((_PER,), f32),
          pltpu.VMEM((_LANES,), f32),
          pltpu.VMEM((_RECS,), f32),
          pltpu.VMEM((_MAX_DET * _LANES,), f32),
          pltpu.VMEM((_LANES,), f32),
          pltpu.VMEM((_LANES,), jnp.int32),
          pltpu.VMEM_SHARED((2 * _RECS,), f32),
          pltpu.SemaphoreType.DMA,
      ],
  )


def kernel(boxes, scores, labels):
  ob, osc, olb = _get_sc_kernel()(boxes, scores, labels)
  return (ob.reshape(_MAX_DET, _LANES)[:, :4], osc, olb)


# speedup vs baseline: 1.1778x; 1.1186x over previous
import functools

import jax
import jax.numpy as jnp
from jax import lax
from jax.experimental import pallas as pl
from jax.experimental.pallas import tpu as pltpu
from jax.experimental.pallas import tpu_sc as plsc

_N = 5000
_LANES = 16
_NTILES = 16
_PER = 320
_STRIDE = 312
_IMG_W = 2048.0
_IMG_H = 2048.0
_IOU_THRESH = 0.5
_SCORE_THRESH = 0.05
_MAX_DET = 4
_MAX_COORD = max(_IMG_W, _IMG_H) + 1.0
_NFLD = 8
_RECS = _NTILES * _NFLD


def _nms_body(x1_h, y1_h, x2_h, y2_h, sc_h, lb_h, ob_h, os_h, ol_h,
              x1v, y1v, x2v, y2v, lbv, x1o, y1o, x2o, y2o, sv,
              recv, rec_all, stg_b, stg_s, stg_l, sh_rec, sem):
  wid = lax.axis_index("s")

  @pl.when(lax.axis_index("c") == 0)
  def _():
    start = wid * _STRIDE
    copies = [
        pltpu.make_async_copy(x1_h.at[pl.ds(start, _PER)], x1v, sem),
        pltpu.make_async_copy(y1_h.at[pl.ds(start, _PER)], y1v, sem),
        pltpu.make_async_copy(x2_h.at[pl.ds(start, _PER)], x2v, sem),
        pltpu.make_async_copy(y2_h.at[pl.ds(start, _PER)], y2v, sem),
        pltpu.make_async_copy(sc_h.at[pl.ds(start, _PER)], sv, sem),
        pltpu.make_async_copy(lb_h.at[pl.ds(start, _PER)], lbv, sem),
    ]
    for c in copies:
      c.start()
    for c in copies:
      c.wait()

    lane = lax.iota(jnp.int32, _LANES)
    neg2 = jnp.full((_LANES,), -2.0, jnp.float32)
    zero_i = jnp.zeros((_LANES,), jnp.int32)

    def pre_body(base, carry):
      m, mi = carry
      sl = pl.ds(base, _LANES)
      bx1 = jnp.minimum(jnp.maximum(x1v[sl], 0.0), _IMG_W)
      by1 = jnp.minimum(jnp.maximum(y1v[sl], 0.0), _IMG_H)
      bx2 = jnp.minimum(jnp.maximum(x2v[sl], 0.0), _IMG_W)
      by2 = jnp.minimum(jnp.maximum(y2v[sl], 0.0), _IMG_H)
      off = lbv[sl].astype(jnp.float32) * _MAX_COORD
      x1o[sl] = bx1 + off
      y1o[sl] = by1 + off
      x2o[sl] = bx2 + off
      y2o[sl] = by2 + off
      raw = sv[sl]
      s = jnp.where(raw > _SCORE_THRESH, raw, -1.0)
      sv[sl] = s
      upd = s > m
      m = jnp.where(upd, s, m)
      mi = jnp.where(upd, lane + base, mi)
      return m, mi

    m, mi = plsc.parallel_loop(
        0, _PER, _LANES, unroll=4, carry=(neg2, zero_i))(pre_body)

    mod4 = jnp.bitwise_and(lane, 3)
    grp4 = lax.shift_right_logical(lane, 2)
    full_box = jnp.where(mod4 <= 1, 0.0, jnp.where(mod4 == 2, _IMG_W, _IMG_H))

    def gat(vec, idx):
      return vec.at[idx].get(mode="promise_in_bounds")

    def local_argmax(m, mi):
      for k in (1, 2, 4, 8):
        idx = jnp.bitwise_xor(lane, k)
        om = gat(m, idx)
        omi = gat(mi, idx)
        take = (om > m) | ((om == m) & (omi < mi))
        m = jnp.where(take, om, m)
        mi = jnp.where(take, omi, mi)
      return m, mi

    ob_vec = jnp.zeros((_LANES,), jnp.float32)
    os_vec = jnp.zeros((_LANES,), jnp.float32)
    ol_vec = jnp.zeros((_LANES,), jnp.int32)

    for d in range(_MAX_DET):
      lm, lli = local_argmax(m, mi)

      par = (d % 2) * _RECS
      gmi = (lli + wid * _STRIDE).astype(jnp.float32)
      lx1 = plsc.load_gather(x1o, [lli])
      ly1 = plsc.load_gather(y1o, [lli])
      lx2 = plsc.load_gather(x2o, [lli])
      ly2 = plsc.load_gather(y2o, [lli])
      llb = plsc.load_gather(lbv, [lli]).astype(jnp.float32)
      rec = jnp.where(lane == 0, lm,
            jnp.where(lane == 1, gmi,
            jnp.where(lane == 2, lx1,
            jnp.where(lane == 3, ly1,
            jnp.where(lane == 4, lx2,
            jnp.where(lane == 5, ly2,
            jnp.where(lane == 6, llb, 0.0)))))))
      recv[...] = rec
      pltpu.sync_copy(recv.at[pl.ds(0, _NFLD)],
                      sh_rec.at[pl.ds(par + wid * _NFLD, _NFLD)])
      plsc.subcore_barrier()
      pltpu.sync_copy(sh_rec.at[pl.ds(par, _RECS)], rec_all)

      wm = plsc.load_gather(rec_all, [lane * _NFLD + 0])
      wmi = plsc.load_gather(rec_all, [lane * _NFLD + 1])
      wtid = lane
      for k in (1, 2, 4, 8):
        idx = jnp.bitwise_xor(lane, k)
        om = gat(wm, idx)
        omi = gat(wmi, idx)
        take = (om > wm) | ((om == wm) & (omi < wmi))
        wm = jnp.where(take, om, wm)
        wmi = jnp.where(take, omi, wmi)
        wtid = jnp.where(take, gat(wtid, idx), wtid)
      wbase = wtid * _NFLD
      wx1 = plsc.load_gather(rec_all, [wbase + 2])
      wy1 = plsc.load_gather(rec_all, [wbase + 3])
      wx2 = plsc.load_gather(rec_all, [wbase + 4])
      wy2 = plsc.load_gather(rec_all, [wbase + 5])
      wlb = plsc.load_gather(rec_all, [wbase + 6])

      ca = (wx2 - wx1) * (wy2 - wy1)
      clbi = wlb.astype(jnp.int32)
      coff = wlb * _MAX_COORD
      cx1 = wx1 - coff
      cy1 = wy1 - coff
      cx2 = wx2 - coff
      cy2 = wy2 - coff

      badv = (((cy2.astype(jnp.int32) - cy1.astype(jnp.int32)) < 1)
              | ((cx2.astype(jnp.int32) - cx1.astype(jnp.int32)) < 1)
              | (wm < 0.0))
      boxsel = jnp.where(mod4 == 0, cx1,
                         jnp.where(mod4 == 1, cy1,
                                   jnp.where(mod4 == 2, cx2, cy2)))
      boxsel = jnp.where(badv, full_box, boxsel)
      ob_vec = jnp.where(grp4 == d, boxsel, ob_vec)
      os_vec = jnp.where(lane == d, jnp.where(wm < 0.0, 0.0, wm), os_vec)
      ol_vec = jnp.where(lane == d, jnp.where(badv, 0, clbi), ol_vec)

      if d + 1 < _MAX_DET:
        def sup_body(base, carry, wx1=wx1, wy1=wy1, wx2=wx2, wy2=wy2, ca=ca):
          m, mi = carry
          sl = pl.ds(base, _LANES)
          xo1 = x1o[sl]
          yo1 = y1o[sl]
          xo2 = x2o[sl]
          yo2 = y2o[sl]
          ltx = jnp.maximum(wx1, xo1)
          lty = jnp.maximum(wy1, yo1)
          rbx = jnp.minimum(wx2, xo2)
          rby = jnp.minimum(wy2, yo2)
          w = jnp.maximum(rbx - ltx, 0.0)
          h = jnp.maximum(rby - lty, 0.0)
          inter = w * h
          area = (xo2 - xo1) * (yo2 - yo1)
          union = jnp.maximum(ca + area - inter, 1e-9)
          s = jnp.where(inter > _IOU_THRESH * union, -1.0, sv[sl])
          sv[sl] = s
          upd = s > m
          m = jnp.where(upd, s, m)
          mi = jnp.where(upd, lane + base, mi)
          return m, mi

        m, mi = plsc.parallel_loop(
            0, _PER, _LANES, unroll=4, carry=(neg2, zero_i))(sup_body)

    @pl.when(wid == 0)
    def _():
      stg_b[...] = ob_vec
      stg_s[...] = os_vec
      stg_l[...] = ol_vec
      outc = [
          pltpu.make_async_copy(stg_b, ob_h, sem),
          pltpu.make_async_copy(stg_s.at[pl.ds(0, _MAX_DET)], os_h, sem),
          pltpu.make_async_copy(stg_l.at[pl.ds(0, _MAX_DET)], ol_h, sem),
      ]
      for c in outc:
        c.start()
      for c in outc:
        c.wait()


@functools.cache
def _get_sc_kernel():
  mesh = plsc.VectorSubcoreMesh(core_axis_name="c", subcore_axis_name="s")
  f32 = jnp.float32
  return pl.kernel(
      _nms_body,
      out_type=(
          jax.ShapeDtypeStruct((_LANES,), f32),
          jax.ShapeDtypeStruct((_MAX_DET,), f32),
          jax.ShapeDtypeStruct((_MAX_DET,), jnp.int32),
      ),
      mesh=mesh,
      compiler_params=pltpu.CompilerParams(needs_layout_passes=False),
      scratch_types=[
          pltpu.VMEM((_PER,), f32),
          pltpu.VMEM((_PER,), f32),
          pltpu.VMEM((_PER,), f32),
          pltpu.VMEM((_PER,), f32),
          pltpu.VMEM((_PER,), jnp.int32),
          pltpu.VMEM((_PER,), f32),
          pltpu.VMEM((_PER,), f32),
          pltpu.VMEM((_PER,), f32),
          pltpu.VMEM((_PER,), f32),
          pltpu.VMEM((_PER,), f32),
          pltpu.VMEM((_LANES,), f32),
          pltpu.VMEM((_RECS,), f32),
          pltpu.VMEM((_LANES,), f32),
          pltpu.VMEM((_LANES,), f32),
          pltpu.VMEM((_LANES,), jnp.int32),
          pltpu.VMEM_SHARED((2 * _RECS,), f32),
          pltpu.SemaphoreType.DMA,
      ],
  )


def kernel(boxes, scores, labels):
  ob, osc, olb = _get_sc_kernel()(
      boxes[:, 0], boxes[:, 1], boxes[:, 2], boxes[:, 3], scores, labels)
  return (ob.reshape(_MAX_DET, 4), osc, olb)
